# Initial kernel scaffold; baseline (speedup 1.0000x reference)
import jax
import jax.numpy as jnp
from jax.experimental import pallas as pl


def kernel(x):
    raise NotImplementedError("write your pallas kernel here")


# speedup vs baseline: 2.0870x; 2.0870x over previous
import functools

import jax
import jax.numpy as jnp
from jax import lax
from jax.experimental import pallas as pl
from jax.experimental.pallas import tpu as pltpu
from jax.experimental.pallas import tpu_sc as plsc

BATCH = 4
SEQ = 4096
CHAN = 1024
KTOP = 64
LANES = 16
NTILES = 32
CGROUPS = CHAN // LANES
NGROUPS = BATCH * CGROUPS
GROUPS_PER_TILE = NGROUPS // NTILES

_MESH = plsc.VectorSubcoreMesh(core_axis_name="c", subcore_axis_name="s")


@functools.partial(
    pl.kernel,
    out_type=jax.ShapeDtypeStruct((BATCH, KTOP, CHAN), jnp.float32),
    mesh=_MESH,
    compiler_params=pltpu.CompilerParams(
        use_tc_tiling_on_sc=False, needs_layout_passes=False
    ),
    scratch_types=[
        pltpu.VMEM((SEQ, LANES), jnp.float32),
        pltpu.VMEM((256, LANES), jnp.int32),
        pltpu.VMEM((KTOP, LANES), jnp.float32),
    ],
)
def _kmax_sc(x_hbm, out_hbm, data_v, hist_v, outb_v):
    wid = lax.axis_index("s") * 2 + lax.axis_index("c")
    lane = lax.iota(jnp.int32, LANES)
    zeros = jnp.zeros((LANES,), jnp.int32)
    ones = jnp.ones((LANES,), jnp.int32)

    def _zero_hist(j, c):
        hist_v[j] = zeros
        return c

    lax.fori_loop(0, 256, _zero_hist, 0)

    def _scan_hist(need):
        def body(t, st):
            cum, fnd, bsel, cbef = st
            j = 255 - t
            h = hist_v[j]
            hist_v[j] = zeros
            cum_new = cum + h
            newly = jnp.logical_and(jnp.logical_not(fnd), cum_new >= need)
            bsel = jnp.where(newly, j, bsel)
            cbef = jnp.where(newly, cum, cbef)
            fnd = jnp.logical_or(fnd, newly)
            return cum_new, fnd, bsel, cbef

        init = (zeros, jnp.zeros((LANES,), jnp.bool_), zeros, zeros)
        _, _, bsel, cbef = lax.fori_loop(0, 256, body, init)
        return bsel, cbef

    def group_body(g, carry):
        gid = g * NTILES + wid
        b = gid // CGROUPS
        c0 = (gid % CGROUPS) * LANES
        pltpu.sync_copy(x_hbm.at[b, :, pl.ds(c0, LANES)], data_v)

        def lvl0(i, c):
            bits = lax.bitcast_convert_type(data_v[i], jnp.int32)
            key = jnp.where(bits < 0, bits ^ 0x7FFFFFFF, bits)
            data_v[i] = lax.bitcast_convert_type(key, jnp.float32)
            su = (key >> 24) + 128
            plsc.addupdate_scatter(hist_v, [su, lane], ones)
            return c

        lax.fori_loop(0, SEQ, lvl0, 0)
        need = jnp.full((LANES,), KTOP, jnp.int32)
        base = jnp.full((LANES,), -128, jnp.int32)
        bsel, cbef = _scan_hist(need)
        prefix = base + bsel
        need = need - cbef

        for lvl in range(1, 4):
            shift = 24 - 8 * lvl
            base = prefix * 256

            def lvlN(i, c, shift=shift, base=base):
                key = lax.bitcast_convert_type(data_v[i], jnp.int32)
                su = (key >> shift) - base
                match = jnp.logical_and(su >= 0, su < 256)
                suc = jnp.clip(su, 0, 255)
                plsc.addupdate_scatter(hist_v, [suc, lane], ones, mask=match)
                return c

            lax.fori_loop(0, SEQ, lvlN, 0)
            bsel, cbef = _scan_hist(need)
            prefix = base + bsel
            need = need - cbef

        tkey = prefix
        rneed = need

        def outp(i, st):
            cnt, eqc = st
            key = lax.bitcast_convert_type(data_v[i], jnp.int32)
            gt = key > tkey
            eq = key == tkey
            sel = jnp.logical_or(gt, jnp.logical_and(eq, eqc < rneed))
            val = jnp.where(key < 0, key ^ 0x7FFFFFFF, key)
            plsc.store_scatter(
                outb_v, [cnt, lane], lax.bitcast_convert_type(val, jnp.float32), mask=sel
            )
            return cnt + sel.astype(jnp.int32), eqc + eq.astype(jnp.int32)

        lax.fori_loop(0, SEQ, outp, (zeros, zeros))
        pltpu.sync_copy(outb_v, out_hbm.at[b, :, pl.ds(c0, LANES)])
        return carry

    lax.fori_loop(0, GROUPS_PER_TILE, group_body, 0)


def kernel(x):
    return _kmax_sc(x)


# speedup vs baseline: 2.2610x; 1.0834x over previous
import functools

import jax
import jax.numpy as jnp
from jax import lax
from jax.experimental import pallas as pl
from jax.experimental.pallas import tpu as pltpu
from jax.experimental.pallas import tpu_sc as plsc

BATCH = 4
SEQ = 4096
CHAN = 1024
KTOP = 64
LANES = 16
NTILES = 32
CGROUPS = CHAN // LANES
NGROUPS = BATCH * CGROUPS
GROUPS_PER_TILE = NGROUPS // NTILES

_MESH = plsc.VectorSubcoreMesh(core_axis_name="c", subcore_axis_name="s")


@functools.partial(
    pl.kernel,
    out_type=jax.ShapeDtypeStruct((BATCH, KTOP, CHAN), jnp.float32),
    mesh=_MESH,
    compiler_params=pltpu.CompilerParams(
        use_tc_tiling_on_sc=False, needs_layout_passes=False
    ),
    scratch_types=[
        pltpu.VMEM((SEQ, LANES), jnp.float32),
        pltpu.VMEM((256, LANES), jnp.int32),
        pltpu.VMEM((KTOP, LANES), jnp.float32),
    ],
)
def _kmax_sc(x_hbm, out_hbm, data_v, hist_v, outb_v):
    wid = lax.axis_index("s") * 2 + lax.axis_index("c")
    lane = lax.iota(jnp.int32, LANES)
    zeros = jnp.zeros((LANES,), jnp.int32)
    ones = jnp.ones((LANES,), jnp.int32)

    def _zero_hist(j, c):
        hist_v[j] = zeros
        return c

    lax.fori_loop(0, 256, _zero_hist, 0)

    def _scan_hist(need):
        def body(t, st):
            cum, fnd, bsel, cbef = st
            for u in range(4):
                j = 255 - (t * 4 + u)
                h = hist_v[j]
                hist_v[j] = zeros
                cum_new = cum + h
                newly = jnp.logical_and(jnp.logical_not(fnd), cum_new >= need)
                bsel = jnp.where(newly, j, bsel)
                cbef = jnp.where(newly, cum, cbef)
                fnd = jnp.logical_or(fnd, newly)
                cum = cum_new
            return cum, fnd, bsel, cbef

        init = (zeros, jnp.zeros((LANES,), jnp.bool_), zeros, zeros)
        _, _, bsel, cbef = lax.fori_loop(0, 64, body, init)
        return bsel, cbef

    def group_body(g, carry):
        gid = g * NTILES + wid
        b = gid // CGROUPS
        c0 = (gid % CGROUPS) * LANES
        pltpu.sync_copy(x_hbm.at[b, :, pl.ds(c0, LANES)], data_v)

        def lvl0(i8, c):
            for u in range(8):
                i = i8 * 8 + u
                bits = lax.bitcast_convert_type(data_v[i], jnp.int32)
                key = jnp.where(bits < 0, bits ^ 0x7FFFFFFF, bits)
                data_v[i] = lax.bitcast_convert_type(key, jnp.float32)
                su = (key >> 24) + 128
                plsc.addupdate_scatter(hist_v, [su, lane], ones)
            return c

        lax.fori_loop(0, SEQ // 8, lvl0, 0)
        need = jnp.full((LANES,), KTOP, jnp.int32)
        base = jnp.full((LANES,), -128, jnp.int32)
        bsel, cbef = _scan_hist(need)
        prefix = base + bsel
        need = need - cbef

        for lvl in range(1, 4):
            shift = 24 - 8 * lvl
            base = prefix * 256

            def lvlN(i8, c, shift=shift, base=base):
                for u in range(8):
                    i = i8 * 8 + u
                    key = lax.bitcast_convert_type(data_v[i], jnp.int32)
                    su = (key >> shift) - base
                    match = jnp.logical_and(su >= 0, su < 256)
                    suc = jnp.clip(su, 0, 255)
                    plsc.addupdate_scatter(hist_v, [suc, lane], ones, mask=match)
                return c

            lax.fori_loop(0, SEQ // 8, lvlN, 0)
            bsel, cbef = _scan_hist(need)
            prefix = base + bsel
            need = need - cbef

        tkey = prefix
        rneed = need

        def outp(i8, st):
            cnt, eqc = st
            for u in range(8):
                i = i8 * 8 + u
                key = lax.bitcast_convert_type(data_v[i], jnp.int32)
                gt = key > tkey
                eq = key == tkey
                sel = jnp.logical_or(gt, jnp.logical_and(eq, eqc < rneed))
                val = jnp.where(key < 0, key ^ 0x7FFFFFFF, key)
                plsc.store_scatter(
                    outb_v,
                    [cnt, lane],
                    lax.bitcast_convert_type(val, jnp.float32),
                    mask=sel,
                )
                cnt = cnt + sel.astype(jnp.int32)
                eqc = eqc + eq.astype(jnp.int32)
            return cnt, eqc

        lax.fori_loop(0, SEQ // 8, outp, (zeros, zeros))
        pltpu.sync_copy(outb_v, out_hbm.at[b, :, pl.ds(c0, LANES)])
        return carry

    lax.fori_loop(0, GROUPS_PER_TILE, group_body, 0)


def kernel(x):
    return _kmax_sc(x)


# speedup vs baseline: 4.6029x; 2.0358x over previous
import functools

import jax
import jax.numpy as jnp
from jax import lax
from jax.experimental import pallas as pl
from jax.experimental.pallas import tpu as pltpu
from jax.experimental.pallas import tpu_sc as plsc

BATCH = 4
SEQ = 4096
CHAN = 1024
KTOP = 64
LANES = 16
NTILES = 32
CGROUPS = CHAN // LANES
NGROUPS = BATCH * CGROUPS
GROUPS_PER_TILE = NGROUPS // NTILES
CAND_CAP = 2048

_MESH = plsc.VectorSubcoreMesh(core_axis_name="c", subcore_axis_name="s")


@functools.partial(
    pl.kernel,
    out_type=jax.ShapeDtypeStruct((BATCH, KTOP, CHAN), jnp.float32),
    mesh=_MESH,
    compiler_params=pltpu.CompilerParams(
        use_tc_tiling_on_sc=False, needs_layout_passes=False
    ),
    scratch_types=[
        pltpu.VMEM((SEQ, LANES), jnp.float32),
        pltpu.VMEM((CAND_CAP, LANES), jnp.int32),
        pltpu.VMEM((256, LANES), jnp.int32),
        pltpu.VMEM((KTOP, LANES), jnp.float32),
    ],
)
def _kmax_sc(x_hbm, out_hbm, data_v, cand_v, hist_v, outb_v):
    wid = lax.axis_index("s") * 2 + lax.axis_index("c")
    lane = lax.iota(jnp.int32, LANES)
    zeros = jnp.zeros((LANES,), jnp.int32)
    ones = jnp.ones((LANES,), jnp.int32)

    def _zero_hist(j, c):
        hist_v[j] = zeros
        return c

    lax.fori_loop(0, 256, _zero_hist, 0)

    def _key(vec_f32):
        bits = lax.bitcast_convert_type(vec_f32, jnp.int32)
        return jnp.where(bits < 0, bits ^ 0x7FFFFFFF, bits)

    def _scan_hist(need):
        def body(t, st):
            cum, fnd, bsel, cbef = st
            for u in range(4):
                j = 255 - (t * 4 + u)
                h = hist_v[j]
                hist_v[j] = zeros
                cum_new = cum + h
                newly = jnp.logical_and(jnp.logical_not(fnd), cum_new >= need)
                bsel = jnp.where(newly, j, bsel)
                cbef = jnp.where(newly, cum, cbef)
                fnd = jnp.logical_or(fnd, newly)
                cum = cum_new
            return cum, fnd, bsel, cbef

        init = (zeros, jnp.zeros((LANES,), jnp.bool_), zeros, zeros)
        _, _, bsel, cbef = lax.fori_loop(0, 64, body, init)
        return bsel, cbef

    def group_body(g, carry):
        gid = g * NTILES + wid
        b = gid // CGROUPS
        c0 = (gid % CGROUPS) * LANES
        pltpu.sync_copy(x_hbm.at[b, :, pl.ds(c0, LANES)], data_v)

        def lvl0(i8, c):
            for u in range(8):
                key = _key(data_v[i8 * 8 + u])
                su = (key >> 24) + 128
                plsc.addupdate_scatter(hist_v, [su, lane], ones)
            return c

        lax.fori_loop(0, SEQ // 8, lvl0, 0)
        need = jnp.full((LANES,), KTOP, jnp.int32)
        b0, cbef = _scan_hist(need)
        prefix = jnp.full((LANES,), -128, jnp.int32) + b0
        need = need - cbef

        base1 = prefix * 256

        def compact(i8, cnt):
            for u in range(8):
                key = _key(data_v[i8 * 8 + u])
                su = (key >> 24) + 128
                m_cand = su >= b0
                slot = jnp.minimum(cnt, CAND_CAP - 1)
                plsc.store_scatter(cand_v, [slot, lane], key, mask=m_cand)
                m_eq = su == b0
                su1 = jnp.clip((key >> 16) - base1, 0, 255)
                plsc.addupdate_scatter(hist_v, [su1, lane], ones, mask=m_eq)
                cnt = cnt + m_cand.astype(jnp.int32)
            return cnt

        cnt = lax.fori_loop(0, SEQ // 8, compact, zeros)
        b1, cbef = _scan_hist(need)
        prefix = base1 + b1
        need = need - cbef

        nblk = (jnp.max(cnt) + 7) // 8

        for shift in (8, 0):
            base = prefix * 256

            def lvlN(i8, c, shift=shift, base=base):
                for u in range(8):
                    i = i8 * 8 + u
                    key = cand_v[i]
                    valid = i < cnt
                    su = (key >> shift) - base
                    match = jnp.logical_and(
                        valid, jnp.logical_and(su >= 0, su < 256)
                    )
                    suc = jnp.clip(su, 0, 255)
                    plsc.addupdate_scatter(hist_v, [suc, lane], ones, mask=match)
                return c

            lax.fori_loop(0, nblk, lvlN, 0)
            bN, cbef = _scan_hist(need)
            prefix = base + bN
            need = need - cbef

        tkey = prefix
        rneed = need

        def outp(i8, st):
            cnt2, eqc = st
            for u in range(8):
                i = i8 * 8 + u
                key = cand_v[i]
                valid = i < cnt
                gt = jnp.logical_and(valid, key > tkey)
                eq = jnp.logical_and(valid, key == tkey)
                sel = jnp.logical_or(gt, jnp.logical_and(eq, eqc < rneed))
                val = jnp.where(key < 0, key ^ 0x7FFFFFFF, key)
                plsc.store_scatter(
                    outb_v,
                    [cnt2, lane],
                    lax.bitcast_convert_type(val, jnp.float32),
                    mask=sel,
                )
                cnt2 = cnt2 + sel.astype(jnp.int32)
                eqc = eqc + eq.astype(jnp.int32)
            return cnt2, eqc

        lax.fori_loop(0, nblk, outp, (zeros, zeros))
        pltpu.sync_copy(outb_v, out_hbm.at[b, :, pl.ds(c0, LANES)])
        return carry

    lax.fori_loop(0, GROUPS_PER_TILE, group_body, 0)


def kernel(x):
    return _kmax_sc(x)


# speedup vs baseline: 4.8256x; 1.0484x over previous
import functools

import jax
import jax.numpy as jnp
from jax import lax
from jax.experimental import pallas as pl
from jax.experimental.pallas import tpu as pltpu
from jax.experimental.pallas import tpu_sc as plsc

BATCH = 4
SEQ = 4096
CHAN = 1024
KTOP = 64
LANES = 16
NTILES = 32
CGROUPS = CHAN // LANES
NGROUPS = BATCH * CGROUPS
GROUPS_PER_TILE = NGROUPS // NTILES
CAND_CAP = 2048

_MESH = plsc.VectorSubcoreMesh(core_axis_name="c", subcore_axis_name="s")


@functools.partial(
    pl.kernel,
    out_type=jax.ShapeDtypeStruct((BATCH, KTOP, CHAN), jnp.float32),
    mesh=_MESH,
    compiler_params=pltpu.CompilerParams(
        use_tc_tiling_on_sc=False, needs_layout_passes=False
    ),
    scratch_types=[
        pltpu.VMEM((SEQ // 2, LANES), jnp.float32),
        pltpu.VMEM((SEQ // 2, LANES), jnp.float32),
        pltpu.VMEM((CAND_CAP, LANES), jnp.int32),
        pltpu.VMEM((256, LANES), jnp.int32),
        pltpu.VMEM((KTOP, LANES), jnp.float32),
        pltpu.SemaphoreType.DMA,
        pltpu.SemaphoreType.DMA,
    ],
)
def _kmax_sc(x_hbm, out_hbm, data_a, data_b, cand_v, hist_v, outb_v, sem_a, sem_b):
    wid = lax.axis_index("s") * 2 + lax.axis_index("c")
    lane = lax.iota(jnp.int32, LANES)
    zeros = jnp.zeros((LANES,), jnp.int32)
    ones = jnp.ones((LANES,), jnp.int32)

    def _zero_hist(j, c):
        hist_v[j] = zeros
        return c

    lax.fori_loop(0, 256, _zero_hist, 0)

    def _key(vec_f32):
        bits = lax.bitcast_convert_type(vec_f32, jnp.int32)
        return jnp.where(bits < 0, bits ^ 0x7FFFFFFF, bits)

    def _scan_hist(need):
        def body(t, st):
            cum, fnd, bsel, cbef = st
            for u in range(4):
                j = 255 - (t * 4 + u)
                h = hist_v[j]
                hist_v[j] = zeros
                cum_new = cum + h
                newly = jnp.logical_and(jnp.logical_not(fnd), cum_new >= need)
                bsel = jnp.where(newly, j, bsel)
                cbef = jnp.where(newly, cum, cbef)
                fnd = jnp.logical_or(fnd, newly)
                cum = cum_new
            return cum, fnd, bsel, cbef

        init = (zeros, jnp.zeros((LANES,), jnp.bool_), zeros, zeros)
        _, _, bsel, cbef = lax.fori_loop(0, 64, body, init)
        return bsel, cbef

    def _addr(g):
        gid = jnp.minimum(g * NTILES + wid, NGROUPS - 1)
        return gid // CGROUPS, (gid % CGROUPS) * LANES

    def _half_copy(g, half, data_ref, sem):
        b, c0 = _addr(g)
        return pltpu.make_async_copy(
            x_hbm.at[b, pl.ds(half * (SEQ // 2), SEQ // 2), pl.ds(c0, LANES)],
            data_ref,
            sem,
        )

    _half_copy(0, 0, data_a, sem_a).start()
    _half_copy(0, 1, data_b, sem_b).start()

    def group_body(g, carry):
        b, c0 = _addr(g)

        def lvl0(data_ref):
            def body(i8, c):
                for u in range(8):
                    key = _key(data_ref[i8 * 8 + u])
                    su = (key >> 24) + 128
                    plsc.addupdate_scatter(hist_v, [su, lane], ones)
                return c

            lax.fori_loop(0, SEQ // 16, body, 0)

        _half_copy(g, 0, data_a, sem_a).wait()
        lvl0(data_a)
        _half_copy(g, 1, data_b, sem_b).wait()
        lvl0(data_b)
        need = jnp.full((LANES,), KTOP, jnp.int32)
        b0, cbef = _scan_hist(need)
        prefix = jnp.full((LANES,), -128, jnp.int32) + b0
        need = need - cbef

        base1 = prefix * 256

        def compact(data_ref, cnt0):
            def body(i8, cnt):
                for u in range(8):
                    key = _key(data_ref[i8 * 8 + u])
                    su = (key >> 24) + 128
                    m_cand = su >= b0
                    slot = jnp.minimum(cnt, CAND_CAP - 1)
                    plsc.store_scatter(cand_v, [slot, lane], key, mask=m_cand)
                    m_eq = su == b0
                    su1 = jnp.clip((key >> 16) - base1, 0, 255)
                    plsc.addupdate_scatter(hist_v, [su1, lane], ones, mask=m_eq)
                    cnt = cnt + m_cand.astype(jnp.int32)
                return cnt

            return lax.fori_loop(0, SEQ // 16, body, cnt0)

        cnt = compact(data_a, zeros)
        _half_copy(g + 1, 0, data_a, sem_a).start()
        cnt = compact(data_b, cnt)
        _half_copy(g + 1, 1, data_b, sem_b).start()
        b1, cbef = _scan_hist(need)
        prefix = base1 + b1
        need = need - cbef

        nblk = (jnp.max(cnt) + 7) // 8

        for shift in (8, 0):
            base = prefix * 256

            def lvlN(i8, c, shift=shift, base=base):
                for u in range(8):
                    i = i8 * 8 + u
                    key = cand_v[i]
                    valid = i < cnt
                    su = (key >> shift) - base
                    match = jnp.logical_and(
                        valid, jnp.logical_and(su >= 0, su < 256)
                    )
                    suc = jnp.clip(su, 0, 255)
                    plsc.addupdate_scatter(hist_v, [suc, lane], ones, mask=match)
                return c

            lax.fori_loop(0, nblk, lvlN, 0)
            bN, cbef = _scan_hist(need)
            prefix = base + bN
            need = need - cbef

        tkey = prefix
        rneed = need

        def outp(i8, st):
            cnt2, eqc = st
            for u in range(8):
                i = i8 * 8 + u
                key = cand_v[i]
                valid = i < cnt
                gt = jnp.logical_and(valid, key > tkey)
                eq = jnp.logical_and(valid, key == tkey)
                sel = jnp.logical_or(gt, jnp.logical_and(eq, eqc < rneed))
                val = jnp.where(key < 0, key ^ 0x7FFFFFFF, key)
                plsc.store_scatter(
                    outb_v,
                    [cnt2, lane],
                    lax.bitcast_convert_type(val, jnp.float32),
                    mask=sel,
                )
                cnt2 = cnt2 + sel.astype(jnp.int32)
                eqc = eqc + eq.astype(jnp.int32)
            return cnt2, eqc

        lax.fori_loop(0, nblk, outp, (zeros, zeros))
        pltpu.sync_copy(outb_v, out_hbm.at[b, :, pl.ds(c0, LANES)])
        return carry

    lax.fori_loop(0, GROUPS_PER_TILE, group_body, 0)

    _half_copy(GROUPS_PER_TILE, 0, data_a, sem_a).wait()
    _half_copy(GROUPS_PER_TILE, 1, data_b, sem_b).wait()


def kernel(x):
    return _kmax_sc(x)


# speedup vs baseline: 12.3803x; 2.5655x over previous
import functools

import jax
import jax.numpy as jnp
from jax import lax
from jax.experimental import pallas as pl
from jax.experimental.pallas import tpu as pltpu
from jax.experimental.pallas import tpu_sc as plsc

BATCH = 4
SEQ = 4096
CHAN = 1024
KTOP = 64
LANES = 16
NTILES = 32
CGROUPS = CHAN // LANES
NGROUPS = BATCH * CGROUPS
GROUPS_PER_TILE = NGROUPS // NTILES
CAND_CAP = 2048

_MESH = plsc.VectorSubcoreMesh(core_axis_name="c", subcore_axis_name="s")


@functools.partial(
    pl.kernel,
    out_type=jax.ShapeDtypeStruct((BATCH, KTOP, CHAN), jnp.float32),
    mesh=_MESH,
    compiler_params=pltpu.CompilerParams(
        use_tc_tiling_on_sc=False, needs_layout_passes=False
    ),
    scratch_types=[
        pltpu.VMEM((SEQ // 2, LANES), jnp.float32),
        pltpu.VMEM((SEQ // 2, LANES), jnp.float32),
        pltpu.VMEM((CAND_CAP, LANES), jnp.int32),
        pltpu.VMEM((256, LANES), jnp.int32),
        pltpu.VMEM((KTOP, LANES), jnp.float32),
        pltpu.SemaphoreType.DMA,
        pltpu.SemaphoreType.DMA,
    ],
)
def _kmax_sc(x_hbm, out_hbm, data_a, data_b, cand_v, hist_v, outb_v, sem_a, sem_b):
    wid = lax.axis_index("s") * 2 + lax.axis_index("c")
    lane = lax.iota(jnp.int32, LANES)
    zeros = jnp.zeros((LANES,), jnp.int32)
    ones = jnp.ones((LANES,), jnp.int32)

    def _zero_hist(j, c):
        hist_v[j] = zeros
        return c

    lax.fori_loop(0, 256, _zero_hist, 0)

    def _key(vec_f32):
        bits = lax.bitcast_convert_type(vec_f32, jnp.int32)
        return jnp.where(bits < 0, bits ^ 0x7FFFFFFF, bits)

    def _scan_hist(need):
        def body(t, st):
            cum, fnd, bsel, cbef = st
            for u in range(4):
                j = 255 - (t * 4 + u)
                h = hist_v[j]
                hist_v[j] = zeros
                cum_new = cum + h
                newly = jnp.logical_and(jnp.logical_not(fnd), cum_new >= need)
                bsel = jnp.where(newly, j, bsel)
                cbef = jnp.where(newly, cum, cbef)
                fnd = jnp.logical_or(fnd, newly)
                cum = cum_new
            return cum, fnd, bsel, cbef

        init = (zeros, jnp.zeros((LANES,), jnp.bool_), zeros, zeros)
        _, _, bsel, cbef = lax.fori_loop(0, 64, body, init)
        return bsel, cbef

    def _addr(g):
        gid = jnp.minimum(g * NTILES + wid, NGROUPS - 1)
        return gid // CGROUPS, (gid % CGROUPS) * LANES

    def _half_copy(g, half, data_ref, sem):
        b, c0 = _addr(g)
        return pltpu.make_async_copy(
            x_hbm.at[b, pl.ds(half * (SEQ // 2), SEQ // 2), pl.ds(c0, LANES)],
            data_ref,
            sem,
        )

    _half_copy(0, 0, data_a, sem_a).start()
    _half_copy(0, 1, data_b, sem_b).start()

    def group_body(g, carry):
        b, c0 = _addr(g)

        def lvl0(data_ref):
            @plsc.parallel_loop(0, SEQ // 2, unroll=8)
            def body(i):
                key = _key(data_ref[i])
                su = (key >> 24) + 128
                plsc.addupdate_scatter(hist_v, [su, lane], ones)

        _half_copy(g, 0, data_a, sem_a).wait()
        lvl0(data_a)
        _half_copy(g, 1, data_b, sem_b).wait()
        lvl0(data_b)
        need = jnp.full((LANES,), KTOP, jnp.int32)
        b0, cbef = _scan_hist(need)
        prefix = jnp.full((LANES,), -128, jnp.int32) + b0
        need = need - cbef

        base1 = prefix * 256

        def compact(data_ref, cnt0):
            @plsc.parallel_loop(0, SEQ // 2, unroll=8, carry=cnt0)
            def body(i, cnt):
                key = _key(data_ref[i])
                su = (key >> 24) + 128
                m_cand = su >= b0
                slot = jnp.minimum(cnt, CAND_CAP - 1)
                plsc.store_scatter(cand_v, [slot, lane], key, mask=m_cand)
                m_eq = su == b0
                su1 = jnp.clip((key >> 16) - base1, 0, 255)
                plsc.addupdate_scatter(hist_v, [su1, lane], ones, mask=m_eq)
                return cnt + m_cand.astype(jnp.int32)

            return body

        cnt = compact(data_a, zeros)
        _half_copy(g + 1, 0, data_a, sem_a).start()
        cnt = compact(data_b, cnt)
        _half_copy(g + 1, 1, data_b, sem_b).start()
        b1, cbef = _scan_hist(need)
        prefix = base1 + b1
        need = need - cbef

        nblk = (jnp.max(cnt) + 7) // 8

        for shift in (8, 0):
            base = prefix * 256

            def lvlN(i8, c, shift=shift, base=base):
                for u in range(8):
                    i = i8 * 8 + u
                    key = cand_v[i]
                    valid = i < cnt
                    su = (key >> shift) - base
                    match = jnp.logical_and(
                        valid, jnp.logical_and(su >= 0, su < 256)
                    )
                    suc = jnp.clip(su, 0, 255)
                    plsc.addupdate_scatter(hist_v, [suc, lane], ones, mask=match)
                return c

            lax.fori_loop(0, nblk, lvlN, 0)
            bN, cbef = _scan_hist(need)
            prefix = base + bN
            need = need - cbef

        tkey = prefix
        rneed = need

        def outp(i8, st):
            cnt2, eqc = st
            for u in range(8):
                i = i8 * 8 + u
                key = cand_v[i]
                valid = i < cnt
                gt = jnp.logical_and(valid, key > tkey)
                eq = jnp.logical_and(valid, key == tkey)
                sel = jnp.logical_or(gt, jnp.logical_and(eq, eqc < rneed))
                val = jnp.where(key < 0, key ^ 0x7FFFFFFF, key)
                plsc.store_scatter(
                    outb_v,
                    [cnt2, lane],
                    lax.bitcast_convert_type(val, jnp.float32),
                    mask=sel,
                )
                cnt2 = cnt2 + sel.astype(jnp.int32)
                eqc = eqc + eq.astype(jnp.int32)
            return cnt2, eqc

        lax.fori_loop(0, nblk, outp, (zeros, zeros))
        pltpu.sync_copy(outb_v, out_hbm.at[b, :, pl.ds(c0, LANES)])
        return carry

    lax.fori_loop(0, GROUPS_PER_TILE, group_body, 0)

    _half_copy(GROUPS_PER_TILE, 0, data_a, sem_a).wait()
    _half_copy(GROUPS_PER_TILE, 1, data_b, sem_b).wait()


def kernel(x):
    return _kmax_sc(x)


# speedup vs baseline: 13.0823x; 1.0567x over previous
import functools

import jax
import jax.numpy as jnp
from jax import lax
from jax.experimental import pallas as pl
from jax.experimental.pallas import tpu as pltpu
from jax.experimental.pallas import tpu_sc as plsc

BATCH = 4
SEQ = 4096
CHAN = 1024
KTOP = 64
LANES = 16
NTILES = 32
CGROUPS = CHAN // LANES
NGROUPS = BATCH * CGROUPS
GROUPS_PER_TILE = NGROUPS // NTILES
CAND_CAP = 2048

_MESH = plsc.VectorSubcoreMesh(core_axis_name="c", subcore_axis_name="s")


@functools.partial(
    pl.kernel,
    out_type=jax.ShapeDtypeStruct((BATCH, KTOP, CHAN), jnp.float32),
    mesh=_MESH,
    compiler_params=pltpu.CompilerParams(
        use_tc_tiling_on_sc=False, needs_layout_passes=False
    ),
    scratch_types=[
        pltpu.VMEM((SEQ // 2, LANES), jnp.float32),
        pltpu.VMEM((SEQ // 2, LANES), jnp.float32),
        pltpu.VMEM((CAND_CAP, LANES), jnp.int32),
        pltpu.VMEM((256, LANES), jnp.int32),
        pltpu.VMEM((KTOP, LANES), jnp.float32),
        pltpu.SemaphoreType.DMA,
        pltpu.SemaphoreType.DMA,
    ],
)
def _kmax_sc(x_hbm, out_hbm, data_a, data_b, cand_v, hist_v, outb_v, sem_a, sem_b):
    wid = lax.axis_index("s") * 2 + lax.axis_index("c")
    lane = lax.iota(jnp.int32, LANES)
    zeros = jnp.zeros((LANES,), jnp.int32)
    ones = jnp.ones((LANES,), jnp.int32)

    def _zero_hist(j, c):
        hist_v[j] = zeros
        return c

    lax.fori_loop(0, 256, _zero_hist, 0)

    def _key(vec_f32):
        bits = lax.bitcast_convert_type(vec_f32, jnp.int32)
        return jnp.where(bits < 0, bits ^ 0x7FFFFFFF, bits)

    def _scan_hist(need):
        init = (zeros, jnp.zeros((LANES,), jnp.bool_), zeros, zeros)

        @plsc.parallel_loop(0, 256, unroll=8, carry=init)
        def body(t, st):
            cum, fnd, bsel, cbef = st
            j = 255 - t
            h = hist_v[j]
            hist_v[j] = zeros
            cum_new = cum + h
            newly = jnp.logical_and(jnp.logical_not(fnd), cum_new >= need)
            bsel = jnp.where(newly, j, bsel)
            cbef = jnp.where(newly, cum, cbef)
            fnd = jnp.logical_or(fnd, newly)
            return cum_new, fnd, bsel, cbef

        _, _, bsel, cbef = body
        return bsel, cbef

    def _addr(g):
        gid = jnp.minimum(g * NTILES + wid, NGROUPS - 1)
        return gid // CGROUPS, (gid % CGROUPS) * LANES

    def _half_copy(g, half, data_ref, sem):
        b, c0 = _addr(g)
        return pltpu.make_async_copy(
            x_hbm.at[b, pl.ds(half * (SEQ // 2), SEQ // 2), pl.ds(c0, LANES)],
            data_ref,
            sem,
        )

    _half_copy(0, 0, data_a, sem_a).start()
    _half_copy(0, 1, data_b, sem_b).start()

    def group_body(g, carry):
        b, c0 = _addr(g)

        def lvl0(data_ref):
            @plsc.parallel_loop(0, SEQ // 2, unroll=8)
            def body(i):
                key = _key(data_ref[i])
                su = (key >> 24) + 128
                plsc.addupdate_scatter(hist_v, [su, lane], ones)

        _half_copy(g, 0, data_a, sem_a).wait()
        lvl0(data_a)
        _half_copy(g, 1, data_b, sem_b).wait()
        lvl0(data_b)
        need = jnp.full((LANES,), KTOP, jnp.int32)
        b0, cbef = _scan_hist(need)
        prefix = jnp.full((LANES,), -128, jnp.int32) + b0
        need = need - cbef

        base1 = prefix * 256

        def compact(data_ref, cnt0):
            @plsc.parallel_loop(0, SEQ // 2, unroll=8, carry=cnt0)
            def body(i, cnt):
                key = _key(data_ref[i])
                su = (key >> 24) + 128
                m_cand = su >= b0
                slot = jnp.minimum(cnt, CAND_CAP - 1)
                plsc.store_scatter(cand_v, [slot, lane], key, mask=m_cand)
                m_eq = su == b0
                su1 = jnp.clip((key >> 16) - base1, 0, 255)
                plsc.addupdate_scatter(hist_v, [su1, lane], ones, mask=m_eq)
                return cnt + m_cand.astype(jnp.int32)

            return body

        cnt = compact(data_a, zeros)
        _half_copy(g + 1, 0, data_a, sem_a).start()
        cnt = compact(data_b, cnt)
        _half_copy(g + 1, 1, data_b, sem_b).start()
        b1, cbef = _scan_hist(need)
        prefix = base1 + b1
        need = need - cbef

        nrows = ((jnp.max(cnt) + 7) // 8) * 8

        for shift in (8, 0):
            base = prefix * 256

            def lvlN(i, shift=shift, base=base):
                key = cand_v[i]
                valid = i < cnt
                su = (key >> shift) - base
                match = jnp.logical_and(
                    valid, jnp.logical_and(su >= 0, su < 256)
                )
                suc = jnp.clip(su, 0, 255)
                plsc.addupdate_scatter(hist_v, [suc, lane], ones, mask=match)

            plsc.parallel_loop(0, nrows, unroll=8)(lvlN)
            bN, cbef = _scan_hist(need)
            prefix = base + bN
            need = need - cbef

        tkey = prefix
        rneed = need

        @plsc.parallel_loop(0, nrows, unroll=8, carry=(zeros, zeros))
        def outp(i, st):
            cnt2, eqc = st
            key = cand_v[i]
            valid = i < cnt
            gt = jnp.logical_and(valid, key > tkey)
            eq = jnp.logical_and(valid, key == tkey)
            sel = jnp.logical_or(gt, jnp.logical_and(eq, eqc < rneed))
            val = jnp.where(key < 0, key ^ 0x7FFFFFFF, key)
            plsc.store_scatter(
                outb_v,
                [cnt2, lane],
                lax.bitcast_convert_type(val, jnp.float32),
                mask=sel,
            )
            return cnt2 + sel.astype(jnp.int32), eqc + eq.astype(jnp.int32)
        pltpu.sync_copy(outb_v, out_hbm.at[b, :, pl.ds(c0, LANES)])
        return carry

    lax.fori_loop(0, GROUPS_PER_TILE, group_body, 0)

    _half_copy(GROUPS_PER_TILE, 0, data_a, sem_a).wait()
    _half_copy(GROUPS_PER_TILE, 1, data_b, sem_b).wait()


def kernel(x):
    return _kmax_sc(x)


# speedup vs baseline: 17.0892x; 1.3063x over previous
import functools

import jax
import jax.numpy as jnp
from jax import lax
from jax.experimental import pallas as pl
from jax.experimental.pallas import tpu as pltpu
from jax.experimental.pallas import tpu_sc as plsc

BATCH = 4
SEQ = 4096
CHAN = 1024
KTOP = 64
LANES = 16
NTILES = 32
CGROUPS = CHAN // LANES
NGROUPS = BATCH * CGROUPS
GROUPS_PER_TILE = NGROUPS // NTILES
CAND_CAP = 2048

_MESH = plsc.VectorSubcoreMesh(core_axis_name="c", subcore_axis_name="s")


@functools.partial(
    pl.kernel,
    out_type=jax.ShapeDtypeStruct((BATCH, KTOP, CHAN), jnp.float32),
    mesh=_MESH,
    compiler_params=pltpu.CompilerParams(
        use_tc_tiling_on_sc=False, needs_layout_passes=False
    ),
    scratch_types=[
        pltpu.VMEM((SEQ // 2, LANES), jnp.float32),
        pltpu.VMEM((SEQ // 2, LANES), jnp.float32),
        pltpu.VMEM((CAND_CAP, LANES), jnp.int32),
        pltpu.VMEM((256, LANES), jnp.int32),
        pltpu.VMEM((KTOP, LANES), jnp.float32),
        pltpu.SemaphoreType.DMA,
        pltpu.SemaphoreType.DMA,
    ],
)
def _kmax_sc(x_hbm, out_hbm, data_a, data_b, cand_v, hist_v, outb_v, sem_a, sem_b):
    wid = lax.axis_index("s") * 2 + lax.axis_index("c")
    lane = lax.iota(jnp.int32, LANES)
    zeros = jnp.zeros((LANES,), jnp.int32)
    ones = jnp.ones((LANES,), jnp.int32)

    def _zero_hist(j, c):
        hist_v[j] = zeros
        return c

    lax.fori_loop(0, 256, _zero_hist, 0)

    def _key(vec_f32):
        bits = lax.bitcast_convert_type(vec_f32, jnp.int32)
        return jnp.where(bits < 0, bits ^ 0x7FFFFFFF, bits)

    def _scan_hist(need):
        init = (zeros, jnp.zeros((LANES,), jnp.bool_), zeros, zeros)

        @plsc.parallel_loop(0, 256, unroll=8, carry=init)
        def body(t, st):
            cum, fnd, bsel, cbef = st
            j = 255 - t
            h = hist_v[j]
            hist_v[j] = zeros
            cum_new = cum + h
            newly = jnp.logical_and(jnp.logical_not(fnd), cum_new >= need)
            bsel = jnp.where(newly, j, bsel)
            cbef = jnp.where(newly, cum, cbef)
            fnd = jnp.logical_or(fnd, newly)
            return cum_new, fnd, bsel, cbef

        _, _, bsel, cbef = body
        return bsel, cbef

    def _addr(g):
        gid = jnp.minimum(g * NTILES + wid, NGROUPS - 1)
        return gid // CGROUPS, (gid % CGROUPS) * LANES

    def _half_copy(g, half, data_ref, sem):
        b, c0 = _addr(g)
        return pltpu.make_async_copy(
            x_hbm.at[b, pl.ds(half * (SEQ // 2), SEQ // 2), pl.ds(c0, LANES)],
            data_ref,
            sem,
        )

    _half_copy(0, 0, data_a, sem_a).start()
    _half_copy(0, 1, data_b, sem_b).start()

    def group_body(g, bguess):
        b, c0 = _addr(g)

        def fused(data_ref, cnt0):
            @plsc.parallel_loop(0, SEQ // 2, unroll=8, carry=cnt0)
            def body(i, cnt):
                key = _key(data_ref[i])
                su = (key >> 24) + 128
                plsc.addupdate_scatter(hist_v, [su, lane], ones)
                m_cand = su >= bguess
                slot = jnp.minimum(cnt, CAND_CAP - 1)
                plsc.store_scatter(cand_v, [slot, lane], key, mask=m_cand)
                return cnt + m_cand.astype(jnp.int32)

            return body

        _half_copy(g, 0, data_a, sem_a).wait()
        cnt = fused(data_a, zeros)
        _half_copy(g, 1, data_b, sem_b).wait()
        cnt = fused(data_b, cnt)
        need = jnp.full((LANES,), KTOP, jnp.int32)
        b0, cbef = _scan_hist(need)
        prefix = jnp.full((LANES,), -128, jnp.int32) + b0
        need = need - cbef

        guess_ok = jnp.logical_and(
            jnp.all(b0 >= bguess), jnp.max(cnt) < CAND_CAP
        )

        def recompact():
            def one(data_ref, cnt0):
                @plsc.parallel_loop(0, SEQ // 2, unroll=8, carry=cnt0)
                def body(i, cnt):
                    key = _key(data_ref[i])
                    m_cand = ((key >> 24) + 128) >= b0
                    slot = jnp.minimum(cnt, CAND_CAP - 1)
                    plsc.store_scatter(cand_v, [slot, lane], key, mask=m_cand)
                    return cnt + m_cand.astype(jnp.int32)

                return body

            return one(data_b, one(data_a, zeros))

        cnt = lax.cond(guess_ok, lambda: cnt, recompact)
        _half_copy(g + 1, 0, data_a, sem_a).start()
        _half_copy(g + 1, 1, data_b, sem_b).start()

        base1 = prefix * 256
        ncand = ((jnp.max(cnt) + 7) // 8) * 8

        def lvl1(i):
            key = cand_v[i]
            valid = i < cnt
            su1 = (key >> 16) - base1
            match = jnp.logical_and(
                valid, jnp.logical_and(su1 >= 0, su1 < 256)
            )
            suc = jnp.clip(su1, 0, 255)
            plsc.addupdate_scatter(hist_v, [suc, lane], ones, mask=match)

        plsc.parallel_loop(0, ncand, unroll=8)(lvl1)
        b1, cbef = _scan_hist(need)
        prefix = base1 + b1
        need = need - cbef
        nrows = ncand

        for shift in (8, 0):
            base = prefix * 256

            def lvlN(i, shift=shift, base=base):
                key = cand_v[i]
                valid = i < cnt
                su = (key >> shift) - base
                match = jnp.logical_and(
                    valid, jnp.logical_and(su >= 0, su < 256)
                )
                suc = jnp.clip(su, 0, 255)
                plsc.addupdate_scatter(hist_v, [suc, lane], ones, mask=match)

            plsc.parallel_loop(0, nrows, unroll=8)(lvlN)
            bN, cbef = _scan_hist(need)
            prefix = base + bN
            need = need - cbef

        tkey = prefix
        rneed = need

        @plsc.parallel_loop(0, nrows, unroll=8, carry=(zeros, zeros))
        def outp(i, st):
            cnt2, eqc = st
            key = cand_v[i]
            valid = i < cnt
            gt = jnp.logical_and(valid, key > tkey)
            eq = jnp.logical_and(valid, key == tkey)
            sel = jnp.logical_or(gt, jnp.logical_and(eq, eqc < rneed))
            val = jnp.where(key < 0, key ^ 0x7FFFFFFF, key)
            plsc.store_scatter(
                outb_v,
                [cnt2, lane],
                lax.bitcast_convert_type(val, jnp.float32),
                mask=sel,
            )
            return cnt2 + sel.astype(jnp.int32), eqc + eq.astype(jnp.int32)
        pltpu.sync_copy(outb_v, out_hbm.at[b, :, pl.ds(c0, LANES)])
        return b0

    lax.fori_loop(
        0, GROUPS_PER_TILE, group_body, jnp.full((LANES,), 255, jnp.int32)
    )

    _half_copy(GROUPS_PER_TILE, 0, data_a, sem_a).wait()
    _half_copy(GROUPS_PER_TILE, 1, data_b, sem_b).wait()


def kernel(x):
    return _kmax_sc(x)


# speedup vs baseline: 17.5734x; 1.0283x over previous
import functools

import jax
import jax.numpy as jnp
from jax import lax
from jax.experimental import pallas as pl
from jax.experimental.pallas import tpu as pltpu
from jax.experimental.pallas import tpu_sc as plsc

BATCH = 4
SEQ = 4096
CHAN = 1024
KTOP = 64
LANES = 16
NTILES = 32
CGROUPS = CHAN // LANES
NGROUPS = BATCH * CGROUPS
GROUPS_PER_TILE = NGROUPS // NTILES
CAND_CAP = 2048

_MESH = plsc.VectorSubcoreMesh(core_axis_name="c", subcore_axis_name="s")


@functools.partial(
    pl.kernel,
    out_type=jax.ShapeDtypeStruct((BATCH, KTOP, CHAN), jnp.float32),
    mesh=_MESH,
    compiler_params=pltpu.CompilerParams(
        use_tc_tiling_on_sc=False, needs_layout_passes=False
    ),
    scratch_types=[
        pltpu.VMEM((SEQ // 2, LANES), jnp.float32),
        pltpu.VMEM((SEQ // 2, LANES), jnp.float32),
        pltpu.VMEM((CAND_CAP, LANES), jnp.int32),
        pltpu.VMEM((256, LANES), jnp.int32),
        pltpu.VMEM((KTOP, LANES), jnp.float32),
        pltpu.SemaphoreType.DMA,
        pltpu.SemaphoreType.DMA,
    ],
)
def _kmax_sc(x_hbm, out_hbm, data_a, data_b, cand_v, hist_v, outb_v, sem_a, sem_b):
    wid = lax.axis_index("s") * 2 + lax.axis_index("c")
    lane = lax.iota(jnp.int32, LANES)
    zeros = jnp.zeros((LANES,), jnp.int32)
    ones = jnp.ones((LANES,), jnp.int32)

    def _zero_hist(j, c):
        hist_v[j] = zeros
        return c

    lax.fori_loop(0, 256, _zero_hist, 0)

    def _key(vec_f32):
        bits = lax.bitcast_convert_type(vec_f32, jnp.int32)
        return jnp.where(bits < 0, bits ^ 0x7FFFFFFF, bits)

    def _scan_hist(need):
        init = (zeros, jnp.zeros((LANES,), jnp.bool_), zeros, zeros)

        @plsc.parallel_loop(0, 256, unroll=8, carry=init)
        def body(t, st):
            cum, fnd, bsel, cbef = st
            j = 255 - t
            h = hist_v[j]
            hist_v[j] = zeros
            cum_new = cum + h
            newly = jnp.logical_and(jnp.logical_not(fnd), cum_new >= need)
            bsel = jnp.where(newly, j, bsel)
            cbef = jnp.where(newly, cum, cbef)
            fnd = jnp.logical_or(fnd, newly)
            return cum_new, fnd, bsel, cbef

        _, fnd, bsel, cbef = body
        return bsel, cbef, fnd

    def _addr(g):
        gid = jnp.minimum(g * NTILES + wid, NGROUPS - 1)
        return gid // CGROUPS, (gid % CGROUPS) * LANES

    def _half_copy(g, half, data_ref, sem):
        b, c0 = _addr(g)
        return pltpu.make_async_copy(
            x_hbm.at[b, pl.ds(half * (SEQ // 2), SEQ // 2), pl.ds(c0, LANES)],
            data_ref,
            sem,
        )

    _half_copy(0, 0, data_a, sem_a).start()
    _half_copy(0, 1, data_b, sem_b).start()

    def group_body(g, bguess):
        b, c0 = _addr(g)

        bg_vec = bguess

        def fused(data_ref, cnt0):
            @plsc.parallel_loop(0, SEQ // 2, unroll=8, carry=cnt0)
            def body(i, cnt):
                key = _key(data_ref[i])
                m_cand = ((key >> 24) + 128) >= bg_vec
                slot = jnp.minimum(cnt, CAND_CAP - 1)
                plsc.store_scatter(cand_v, [slot, lane], key, mask=m_cand)
                return cnt + m_cand.astype(jnp.int32)

            return body

        _half_copy(g, 0, data_a, sem_a).wait()
        cnt = fused(data_a, zeros)
        _half_copy(g, 1, data_b, sem_b).wait()
        cnt = fused(data_b, cnt)

        def candhist(cnt_in):
            ncand0 = ((jnp.max(cnt_in) + 7) // 8) * 8

            def body(i):
                key = cand_v[i]
                valid = i < cnt_in
                su = (key >> 24) + 128
                plsc.addupdate_scatter(hist_v, [su, lane], ones, mask=valid)

            plsc.parallel_loop(0, ncand0, unroll=8)(body)

        candhist(cnt)
        need = jnp.full((LANES,), KTOP, jnp.int32)
        b0, cbef, fnd = _scan_hist(need)

        guess_ok = jnp.logical_and(jnp.all(fnd), jnp.max(cnt) < CAND_CAP)

        def fallback():
            def histpass(data_ref):
                @plsc.parallel_loop(0, SEQ // 2, unroll=8)
                def body(i):
                    su = (_key(data_ref[i]) >> 24) + 128
                    plsc.addupdate_scatter(hist_v, [su, lane], ones)

            histpass(data_a)
            histpass(data_b)
            b0f, cbeff, _ = _scan_hist(need)

            def one(data_ref, cnt0):
                @plsc.parallel_loop(0, SEQ // 2, unroll=8, carry=cnt0)
                def body(i, cnt):
                    key = _key(data_ref[i])
                    m_cand = ((key >> 24) + 128) >= b0f
                    slot = jnp.minimum(cnt, CAND_CAP - 1)
                    plsc.store_scatter(cand_v, [slot, lane], key, mask=m_cand)
                    return cnt + m_cand.astype(jnp.int32)

                return body

            return one(data_b, one(data_a, zeros)), b0f, cbeff

        cnt, b0, cbef = lax.cond(
            guess_ok, lambda: (cnt, b0, cbef), fallback
        )
        prefix = jnp.full((LANES,), -128, jnp.int32) + b0
        need = need - cbef
        _half_copy(g + 1, 0, data_a, sem_a).start()
        _half_copy(g + 1, 1, data_b, sem_b).start()

        base1 = prefix * 256
        ncand = ((jnp.max(cnt) + 7) // 8) * 8

        def lvl1(i):
            key = cand_v[i]
            valid = i < cnt
            su1 = (key >> 16) - base1
            match = jnp.logical_and(
                valid, jnp.logical_and(su1 >= 0, su1 < 256)
            )
            suc = jnp.clip(su1, 0, 255)
            plsc.addupdate_scatter(hist_v, [suc, lane], ones, mask=match)

        plsc.parallel_loop(0, ncand, unroll=8)(lvl1)
        b1, cbef, _ = _scan_hist(need)
        prefix = base1 + b1
        need = need - cbef
        nrows = ncand

        for shift in (8, 0):
            base = prefix * 256

            def lvlN(i, shift=shift, base=base):
                key = cand_v[i]
                valid = i < cnt
                su = (key >> shift) - base
                match = jnp.logical_and(
                    valid, jnp.logical_and(su >= 0, su < 256)
                )
                suc = jnp.clip(su, 0, 255)
                plsc.addupdate_scatter(hist_v, [suc, lane], ones, mask=match)

            plsc.parallel_loop(0, nrows, unroll=8)(lvlN)
            bN, cbef, _ = _scan_hist(need)
            prefix = base + bN
            need = need - cbef

        tkey = prefix
        rneed = need

        @plsc.parallel_loop(0, nrows, unroll=8, carry=(zeros, zeros))
        def outp(i, st):
            cnt2, eqc = st
            key = cand_v[i]
            valid = i < cnt
            gt = jnp.logical_and(valid, key > tkey)
            eq = jnp.logical_and(valid, key == tkey)
            sel = jnp.logical_or(gt, jnp.logical_and(eq, eqc < rneed))
            val = jnp.where(key < 0, key ^ 0x7FFFFFFF, key)
            plsc.store_scatter(
                outb_v,
                [cnt2, lane],
                lax.bitcast_convert_type(val, jnp.float32),
                mask=sel,
            )
            return cnt2 + sel.astype(jnp.int32), eqc + eq.astype(jnp.int32)
        pltpu.sync_copy(outb_v, out_hbm.at[b, :, pl.ds(c0, LANES)])
        return b0

    lax.fori_loop(
        0, GROUPS_PER_TILE, group_body, jnp.full((LANES,), 255, jnp.int32)
    )

    _half_copy(GROUPS_PER_TILE, 0, data_a, sem_a).wait()
    _half_copy(GROUPS_PER_TILE, 1, data_b, sem_b).wait()


def kernel(x):
    return _kmax_sc(x)


# speedup vs baseline: 17.7338x; 1.0091x over previous
import functools

import jax
import jax.numpy as jnp
from jax import lax
from jax.experimental import pallas as pl
from jax.experimental.pallas import tpu as pltpu
from jax.experimental.pallas import tpu_sc as plsc

BATCH = 4
SEQ = 4096
CHAN = 1024
KTOP = 64
LANES = 16
NTILES = 32
CGROUPS = CHAN // LANES
NGROUPS = BATCH * CGROUPS
GROUPS_PER_TILE = NGROUPS // NTILES
CAND_CAP = 2048

_MESH = plsc.VectorSubcoreMesh(core_axis_name="c", subcore_axis_name="s")


@functools.partial(
    pl.kernel,
    out_type=jax.ShapeDtypeStruct((BATCH, KTOP, CHAN), jnp.float32),
    mesh=_MESH,
    compiler_params=pltpu.CompilerParams(
        use_tc_tiling_on_sc=False, needs_layout_passes=False
    ),
    scratch_types=[
        pltpu.VMEM((SEQ // 2, LANES), jnp.float32),
        pltpu.VMEM((SEQ // 2, LANES), jnp.float32),
        pltpu.VMEM((CAND_CAP, LANES), jnp.int32),
        pltpu.VMEM((256, LANES), jnp.int32),
        pltpu.VMEM((KTOP, LANES), jnp.float32),
        pltpu.SemaphoreType.DMA,
        pltpu.SemaphoreType.DMA,
    ],
)
def _kmax_sc(x_hbm, out_hbm, data_a, data_b, cand_v, hist_v, outb_v, sem_a, sem_b):
    wid = lax.axis_index("s") * 2 + lax.axis_index("c")
    lane = lax.iota(jnp.int32, LANES)
    zeros = jnp.zeros((LANES,), jnp.int32)
    ones = jnp.ones((LANES,), jnp.int32)

    def _zero_hist(j, c):
        hist_v[j] = zeros
        return c

    lax.fori_loop(0, 256, _zero_hist, 0)

    def _key(vec_f32):
        bits = lax.bitcast_convert_type(vec_f32, jnp.int32)
        return jnp.where(bits < 0, bits ^ 0x7FFFFFFF, bits)

    def _scan_hist(need):
        init = (zeros, jnp.zeros((LANES,), jnp.bool_), zeros, zeros)

        @plsc.parallel_loop(0, 256, unroll=8, carry=init)
        def body(t, st):
            cum, fnd, bsel, cbef = st
            j = 255 - t
            h = hist_v[j]
            hist_v[j] = zeros
            cum_new = cum + h
            newly = jnp.logical_and(jnp.logical_not(fnd), cum_new >= need)
            bsel = jnp.where(newly, j, bsel)
            cbef = jnp.where(newly, cum, cbef)
            fnd = jnp.logical_or(fnd, newly)
            return cum_new, fnd, bsel, cbef

        _, fnd, bsel, cbef = body
        return bsel, cbef, fnd

    def _addr(g):
        gid = jnp.minimum(g * NTILES + wid, NGROUPS - 1)
        return gid // CGROUPS, (gid % CGROUPS) * LANES

    def _half_copy(g, half, data_ref, sem):
        b, c0 = _addr(g)
        return pltpu.make_async_copy(
            x_hbm.at[b, pl.ds(half * (SEQ // 2), SEQ // 2), pl.ds(c0, LANES)],
            data_ref,
            sem,
        )

    def _half_start(g, half, data_ref, sem):
        b, c0 = _addr(g)
        q = SEQ // 4
        for sub in range(2):
            pltpu.make_async_copy(
                x_hbm.at[
                    b,
                    pl.ds(half * (SEQ // 2) + sub * q, q),
                    pl.ds(c0, LANES),
                ],
                data_ref.at[pl.ds(sub * q, q)],
                sem,
            ).start()

    _half_start(0, 0, data_a, sem_a)
    _half_start(0, 1, data_b, sem_b)

    def group_body(g, bguess):
        b, c0 = _addr(g)

        bg_vec = bguess

        def fused(data_ref, cnt0):
            @plsc.parallel_loop(0, SEQ // 2, unroll=8, carry=cnt0)
            def body(i, cnt):
                key = _key(data_ref[i])
                m_cand = ((key >> 24) + 128) >= bg_vec
                slot = jnp.minimum(cnt, CAND_CAP - 1)
                plsc.store_scatter(cand_v, [slot, lane], key, mask=m_cand)
                return cnt + m_cand.astype(jnp.int32)

            return body

        _half_copy(g, 0, data_a, sem_a).wait()
        cnt = fused(data_a, zeros)
        _half_copy(g, 1, data_b, sem_b).wait()
        cnt = fused(data_b, cnt)

        def candhist(cnt_in):
            ncand0 = ((jnp.max(cnt_in) + 7) // 8) * 8

            def body(i):
                key = cand_v[i]
                valid = i < cnt_in
                su = (key >> 24) + 128
                plsc.addupdate_scatter(hist_v, [su, lane], ones, mask=valid)

            plsc.parallel_loop(0, ncand0, unroll=8)(body)

        candhist(cnt)
        need = jnp.full((LANES,), KTOP, jnp.int32)
        b0, cbef, fnd = _scan_hist(need)

        guess_ok = jnp.logical_and(jnp.all(fnd), jnp.max(cnt) < CAND_CAP)

        def fallback():
            def histpass(data_ref):
                @plsc.parallel_loop(0, SEQ // 2, unroll=8)
                def body(i):
                    su = (_key(data_ref[i]) >> 24) + 128
                    plsc.addupdate_scatter(hist_v, [su, lane], ones)

            histpass(data_a)
            histpass(data_b)
            b0f, cbeff, _ = _scan_hist(need)

            def one(data_ref, cnt0):
                @plsc.parallel_loop(0, SEQ // 2, unroll=8, carry=cnt0)
                def body(i, cnt):
                    key = _key(data_ref[i])
                    m_cand = ((key >> 24) + 128) >= b0f
                    slot = jnp.minimum(cnt, CAND_CAP - 1)
                    plsc.store_scatter(cand_v, [slot, lane], key, mask=m_cand)
                    return cnt + m_cand.astype(jnp.int32)

                return body

            return one(data_b, one(data_a, zeros)), b0f, cbeff

        cnt, b0, cbef = lax.cond(
            guess_ok, lambda: (cnt, b0, cbef), fallback
        )
        prefix = jnp.full((LANES,), -128, jnp.int32) + b0
        need = need - cbef
        _half_start(g + 1, 0, data_a, sem_a)
        _half_start(g + 1, 1, data_b, sem_b)

        base1 = prefix * 256
        ncand = ((jnp.max(cnt) + 7) // 8) * 8

        def lvl1(i):
            key = cand_v[i]
            valid = i < cnt
            su1 = (key >> 16) - base1
            match = jnp.logical_and(
                valid, jnp.logical_and(su1 >= 0, su1 < 256)
            )
            suc = jnp.clip(su1, 0, 255)
            plsc.addupdate_scatter(hist_v, [suc, lane], ones, mask=match)

        plsc.parallel_loop(0, ncand, unroll=8)(lvl1)
        b1, cbef, _ = _scan_hist(need)
        prefix = base1 + b1
        need = need - cbef
        nrows = ncand

        for shift in (8, 0):
            base = prefix * 256

            def lvlN(i, shift=shift, base=base):
                key = cand_v[i]
                valid = i < cnt
                su = (key >> shift) - base
                match = jnp.logical_and(
                    valid, jnp.logical_and(su >= 0, su < 256)
                )
                suc = jnp.clip(su, 0, 255)
                plsc.addupdate_scatter(hist_v, [suc, lane], ones, mask=match)

            plsc.parallel_loop(0, nrows, unroll=8)(lvlN)
            bN, cbef, _ = _scan_hist(need)
            prefix = base + bN
            need = need - cbef

        tkey = prefix
        rneed = need

        @plsc.parallel_loop(0, nrows, unroll=8, carry=(zeros, zeros))
        def outp(i, st):
            cnt2, eqc = st
            key = cand_v[i]
            valid = i < cnt
            gt = jnp.logical_and(valid, key > tkey)
            eq = jnp.logical_and(valid, key == tkey)
            sel = jnp.logical_or(gt, jnp.logical_and(eq, eqc < rneed))
            val = jnp.where(key < 0, key ^ 0x7FFFFFFF, key)
            plsc.store_scatter(
                outb_v,
                [cnt2, lane],
                lax.bitcast_convert_type(val, jnp.float32),
                mask=sel,
            )
            return cnt2 + sel.astype(jnp.int32), eqc + eq.astype(jnp.int32)
        pltpu.sync_copy(outb_v, out_hbm.at[b, :, pl.ds(c0, LANES)])
        return b0

    lax.fori_loop(
        0, GROUPS_PER_TILE, group_body, jnp.full((LANES,), 255, jnp.int32)
    )

    _half_copy(GROUPS_PER_TILE, 0, data_a, sem_a).wait()
    _half_copy(GROUPS_PER_TILE, 1, data_b, sem_b).wait()


def kernel(x):
    return _kmax_sc(x)


# speedup vs baseline: 20.0674x; 1.1316x over previous
import functools

import jax
import jax.numpy as jnp
from jax import lax
from jax.experimental import pallas as pl
from jax.experimental.pallas import tpu as pltpu
from jax.experimental.pallas import tpu_sc as plsc

BATCH = 4
SEQ = 4096
CHAN = 1024
KTOP = 64
LANES = 16
NTILES = 32
CGROUPS = CHAN // LANES
NGROUPS = BATCH * CGROUPS
GROUPS_PER_TILE = NGROUPS // NTILES
CAND_CAP = 2048

_MESH = plsc.VectorSubcoreMesh(core_axis_name="c", subcore_axis_name="s")


@functools.partial(
    pl.kernel,
    out_type=jax.ShapeDtypeStruct((BATCH, KTOP, CHAN), jnp.float32),
    mesh=_MESH,
    compiler_params=pltpu.CompilerParams(
        use_tc_tiling_on_sc=False, needs_layout_passes=False
    ),
    scratch_types=[
        pltpu.VMEM((SEQ // 2, LANES), jnp.float32),
        pltpu.VMEM((SEQ // 2, LANES), jnp.float32),
        pltpu.VMEM((CAND_CAP, LANES), jnp.int32),
        pltpu.VMEM((256, LANES), jnp.int32),
        pltpu.VMEM((KTOP, LANES), jnp.float32),
        pltpu.SemaphoreType.DMA,
        pltpu.SemaphoreType.DMA,
    ],
)
def _kmax_sc(x_hbm, out_hbm, data_a, data_b, cand_v, hist_v, outb_v, sem_a, sem_b):
    wid = lax.axis_index("s") * 2 + lax.axis_index("c")
    lane = lax.iota(jnp.int32, LANES)
    zeros = jnp.zeros((LANES,), jnp.int32)
    ones = jnp.ones((LANES,), jnp.int32)

    def _zero_hist(j, c):
        hist_v[j] = zeros
        return c

    lax.fori_loop(0, 256, _zero_hist, 0)

    def _key(vec_f32):
        bits = lax.bitcast_convert_type(vec_f32, jnp.int32)
        return jnp.where(bits < 0, bits ^ 0x7FFFFFFF, bits)

    def _scan_hist(need):
        init = (zeros, jnp.zeros((LANES,), jnp.bool_), zeros, zeros)

        @plsc.parallel_loop(0, 256, unroll=8, carry=init)
        def body(t, st):
            cum, fnd, bsel, cbef = st
            j = 255 - t
            h = hist_v[j]
            hist_v[j] = zeros
            cum_new = cum + h
            newly = jnp.logical_and(jnp.logical_not(fnd), cum_new >= need)
            bsel = jnp.where(newly, j, bsel)
            cbef = jnp.where(newly, cum, cbef)
            fnd = jnp.logical_or(fnd, newly)
            return cum_new, fnd, bsel, cbef

        _, fnd, bsel, cbef = body
        return bsel, cbef, fnd

    def _addr(g):
        gid = jnp.minimum(g * NTILES + wid, NGROUPS - 1)
        return gid // CGROUPS, (gid % CGROUPS) * LANES

    def _half_copy(g, half, data_ref, sem):
        b, c0 = _addr(g)
        return pltpu.make_async_copy(
            x_hbm.at[b, pl.ds(half * (SEQ // 2), SEQ // 2), pl.ds(c0, LANES)],
            data_ref,
            sem,
        )

    def _half_start(g, half, data_ref, sem):
        b, c0 = _addr(g)
        q = SEQ // 4
        for sub in range(2):
            pltpu.make_async_copy(
                x_hbm.at[
                    b,
                    pl.ds(half * (SEQ // 2) + sub * q, q),
                    pl.ds(c0, LANES),
                ],
                data_ref.at[pl.ds(sub * q, q)],
                sem,
            ).start()

    _half_start(0, 0, data_a, sem_a)
    _half_start(0, 1, data_b, sem_b)

    def group_body(g, bguess):
        b, c0 = _addr(g)

        kbase = jnp.maximum(bguess - 128, 0) << 24
        flo = lax.bitcast_convert_type(kbase, jnp.float32)

        def fused(data_ref, cnt0):
            @plsc.parallel_loop(0, SEQ // 2, unroll=8, carry=cnt0)
            def body(i, cnt):
                x = data_ref[i]
                m_cand = x >= flo
                slot = jnp.minimum(cnt, CAND_CAP - 1)
                plsc.store_scatter(
                    cand_v,
                    [slot, lane],
                    lax.bitcast_convert_type(x, jnp.int32),
                    mask=m_cand,
                )
                return cnt + m_cand.astype(jnp.int32)

            return body

        _half_copy(g, 0, data_a, sem_a).wait()
        cnt = fused(data_a, zeros)
        _half_copy(g, 1, data_b, sem_b).wait()
        cnt = fused(data_b, cnt)

        def candhist(cnt_in):
            ncand0 = ((jnp.max(cnt_in) + 7) // 8) * 8

            def body(i):
                raw = cand_v[i]
                key = jnp.where(raw < 0, raw ^ 0x7FFFFFFF, raw)
                cand_v[i] = key
                valid = i < cnt_in
                su = (key >> 24) + 128
                plsc.addupdate_scatter(hist_v, [su, lane], ones, mask=valid)

            plsc.parallel_loop(0, ncand0, unroll=8)(body)

        candhist(cnt)
        need = jnp.full((LANES,), KTOP, jnp.int32)
        b0, cbef, fnd = _scan_hist(need)

        guess_ok = jnp.logical_and(jnp.all(fnd), jnp.max(cnt) < CAND_CAP)

        def fallback():
            def histpass(data_ref):
                @plsc.parallel_loop(0, SEQ // 2, unroll=8)
                def body(i):
                    su = (_key(data_ref[i]) >> 24) + 128
                    plsc.addupdate_scatter(hist_v, [su, lane], ones)

            histpass(data_a)
            histpass(data_b)
            b0f, cbeff, _ = _scan_hist(need)

            def one(data_ref, cnt0):
                @plsc.parallel_loop(0, SEQ // 2, unroll=8, carry=cnt0)
                def body(i, cnt):
                    key = _key(data_ref[i])
                    m_cand = ((key >> 24) + 128) >= b0f
                    slot = jnp.minimum(cnt, CAND_CAP - 1)
                    plsc.store_scatter(cand_v, [slot, lane], key, mask=m_cand)
                    return cnt + m_cand.astype(jnp.int32)

                return body

            return one(data_b, one(data_a, zeros)), b0f, cbeff

        cnt, b0, cbef = lax.cond(
            guess_ok, lambda: (cnt, b0, cbef), fallback
        )
        prefix = jnp.full((LANES,), -128, jnp.int32) + b0
        need = need - cbef
        _half_start(g + 1, 0, data_a, sem_a)
        _half_start(g + 1, 1, data_b, sem_b)

        base1 = prefix * 256
        ncand = ((jnp.max(cnt) + 7) // 8) * 8

        def lvl1(i):
            key = cand_v[i]
            valid = i < cnt
            su1 = (key >> 16) - base1
            match = jnp.logical_and(
                valid, jnp.logical_and(su1 >= 0, su1 < 256)
            )
            suc = jnp.clip(su1, 0, 255)
            plsc.addupdate_scatter(hist_v, [suc, lane], ones, mask=match)

        plsc.parallel_loop(0, ncand, unroll=8)(lvl1)
        b1, cbef, _ = _scan_hist(need)
        prefix = base1 + b1
        need = need - cbef
        nrows = ncand

        for shift in (8, 0):
            base = prefix * 256

            def lvlN(i, shift=shift, base=base):
                key = cand_v[i]
                valid = i < cnt
                su = (key >> shift) - base
                match = jnp.logical_and(
                    valid, jnp.logical_and(su >= 0, su < 256)
                )
                suc = jnp.clip(su, 0, 255)
                plsc.addupdate_scatter(hist_v, [suc, lane], ones, mask=match)

            plsc.parallel_loop(0, nrows, unroll=8)(lvlN)
            bN, cbef, _ = _scan_hist(need)
            prefix = base + bN
            need = need - cbef

        tkey = prefix
        rneed = need

        @plsc.parallel_loop(0, nrows, unroll=8, carry=(zeros, zeros))
        def outp(i, st):
            cnt2, eqc = st
            key = cand_v[i]
            valid = i < cnt
            gt = jnp.logical_and(valid, key > tkey)
            eq = jnp.logical_and(valid, key == tkey)
            sel = jnp.logical_or(gt, jnp.logical_and(eq, eqc < rneed))
            val = jnp.where(key < 0, key ^ 0x7FFFFFFF, key)
            plsc.store_scatter(
                outb_v,
                [cnt2, lane],
                lax.bitcast_convert_type(val, jnp.float32),
                mask=sel,
            )
            return cnt2 + sel.astype(jnp.int32), eqc + eq.astype(jnp.int32)
        pltpu.sync_copy(outb_v, out_hbm.at[b, :, pl.ds(c0, LANES)])
        return b0

    lax.fori_loop(
        0, GROUPS_PER_TILE, group_body, jnp.full((LANES,), 255, jnp.int32)
    )

    _half_copy(GROUPS_PER_TILE, 0, data_a, sem_a).wait()
    _half_copy(GROUPS_PER_TILE, 1, data_b, sem_b).wait()


def kernel(x):
    return _kmax_sc(x)


# speedup vs baseline: 21.8315x; 1.0879x over previous
import functools

import jax
import jax.numpy as jnp
from jax import lax
from jax.experimental import pallas as pl
from jax.experimental.pallas import tpu as pltpu
from jax.experimental.pallas import tpu_sc as plsc

BATCH = 4
SEQ = 4096
CHAN = 1024
KTOP = 64
LANES = 16
NTILES = 32
CGROUPS = CHAN // LANES
NGROUPS = BATCH * CGROUPS
GROUPS_PER_TILE = NGROUPS // NTILES
CAND_CAP = 2048

_MESH = plsc.VectorSubcoreMesh(core_axis_name="c", subcore_axis_name="s")


@functools.partial(
    pl.kernel,
    out_type=jax.ShapeDtypeStruct((BATCH, KTOP, CHAN), jnp.float32),
    mesh=_MESH,
    compiler_params=pltpu.CompilerParams(
        use_tc_tiling_on_sc=False, needs_layout_passes=False
    ),
    scratch_types=[
        pltpu.VMEM((SEQ // 2, LANES), jnp.float32),
        pltpu.VMEM((SEQ // 2, LANES), jnp.float32),
        pltpu.VMEM((CAND_CAP, LANES), jnp.int32),
        pltpu.VMEM((256, LANES), jnp.int32),
        pltpu.VMEM((2 * KTOP, LANES), jnp.float32),
        pltpu.SemaphoreType.DMA,
        pltpu.SemaphoreType.DMA,
        pltpu.SemaphoreType.DMA,
    ],
)
def _kmax_sc(
    x_hbm, out_hbm, data_a, data_b, cand_v, hist_v, outb_v, sem_a, sem_b, sem_o
):
    wid = lax.axis_index("s") * 2 + lax.axis_index("c")
    lane = lax.iota(jnp.int32, LANES)
    zeros = jnp.zeros((LANES,), jnp.int32)
    ones = jnp.ones((LANES,), jnp.int32)

    def _zero_hist(j, c):
        hist_v[j] = zeros
        return c

    lax.fori_loop(0, 256, _zero_hist, 0)

    def _key(vec_f32):
        bits = lax.bitcast_convert_type(vec_f32, jnp.int32)
        return jnp.where(bits < 0, bits ^ 0x7FFFFFFF, bits)

    def _scan_hist(need):
        init = (zeros, jnp.zeros((LANES,), jnp.bool_), zeros, zeros)

        @plsc.parallel_loop(0, 256, unroll=8, carry=init)
        def body(t, st):
            cum, fnd, bsel, cbef = st
            j = 255 - t
            h = hist_v[j]
            hist_v[j] = zeros
            cum_new = cum + h
            newly = jnp.logical_and(jnp.logical_not(fnd), cum_new >= need)
            bsel = jnp.where(newly, j, bsel)
            cbef = jnp.where(newly, cum, cbef)
            fnd = jnp.logical_or(fnd, newly)
            return cum_new, fnd, bsel, cbef

        _, fnd, bsel, cbef = body
        return bsel, cbef, fnd

    def _addr(g):
        gid = jnp.minimum(g * NTILES + wid, NGROUPS - 1)
        return gid // CGROUPS, (gid % CGROUPS) * LANES

    def _half_copy(g, half, data_ref, sem):
        b, c0 = _addr(g)
        return pltpu.make_async_copy(
            x_hbm.at[b, pl.ds(half * (SEQ // 2), SEQ // 2), pl.ds(c0, LANES)],
            data_ref,
            sem,
        )

    def _half_start(g, half, data_ref, sem):
        b, c0 = _addr(g)
        q = SEQ // 4
        for sub in range(2):
            pltpu.make_async_copy(
                x_hbm.at[
                    b,
                    pl.ds(half * (SEQ // 2) + sub * q, q),
                    pl.ds(c0, LANES),
                ],
                data_ref.at[pl.ds(sub * q, q)],
                sem,
            ).start()

    _half_start(0, 0, data_a, sem_a)
    _half_start(0, 1, data_b, sem_b)

    def _out_copy(g):
        b, c0 = _addr(g)
        off = (g % 2) * KTOP
        return pltpu.make_async_copy(
            outb_v.at[pl.ds(off, KTOP)],
            out_hbm.at[b, :, pl.ds(c0, LANES)],
            sem_o,
        )

    def group_body(g, bguess):
        b, c0 = _addr(g)

        @pl.when(g >= 2)
        def _():
            _out_copy(g - 2).wait()

        kbase = jnp.maximum(bguess - 128, 0) << 24
        flo = lax.bitcast_convert_type(kbase, jnp.float32)

        def fused(data_ref, cnt0):
            @plsc.parallel_loop(0, SEQ // 2, unroll=8, carry=cnt0)
            def body(i, cnt):
                x = data_ref[i]
                m_cand = x >= flo
                slot = jnp.minimum(cnt, CAND_CAP - 1)
                plsc.store_scatter(
                    cand_v,
                    [slot, lane],
                    lax.bitcast_convert_type(x, jnp.int32),
                    mask=m_cand,
                )
                return cnt + m_cand.astype(jnp.int32)

            return body

        _half_copy(g, 0, data_a, sem_a).wait()
        cnt = fused(data_a, zeros)
        _half_copy(g, 1, data_b, sem_b).wait()
        cnt = fused(data_b, cnt)

        def candhist(cnt_in):
            ncand0 = ((jnp.max(cnt_in) + 7) // 8) * 8

            def body(i):
                raw = cand_v[i]
                key = jnp.where(raw < 0, raw ^ 0x7FFFFFFF, raw)
                cand_v[i] = key
                valid = i < cnt_in
                su = (key >> 24) + 128
                plsc.addupdate_scatter(hist_v, [su, lane], ones, mask=valid)

            plsc.parallel_loop(0, ncand0, unroll=8)(body)

        candhist(cnt)
        need = jnp.full((LANES,), KTOP, jnp.int32)
        b0, cbef, fnd = _scan_hist(need)

        guess_ok = jnp.logical_and(jnp.all(fnd), jnp.max(cnt) < CAND_CAP)

        def fallback():
            def histpass(data_ref):
                @plsc.parallel_loop(0, SEQ // 2, unroll=8)
                def body(i):
                    su = (_key(data_ref[i]) >> 24) + 128
                    plsc.addupdate_scatter(hist_v, [su, lane], ones)

            histpass(data_a)
            histpass(data_b)
            b0f, cbeff, _ = _scan_hist(need)

            def one(data_ref, cnt0):
                @plsc.parallel_loop(0, SEQ // 2, unroll=8, carry=cnt0)
                def body(i, cnt):
                    key = _key(data_ref[i])
                    m_cand = ((key >> 24) + 128) >= b0f
                    slot = jnp.minimum(cnt, CAND_CAP - 1)
                    plsc.store_scatter(cand_v, [slot, lane], key, mask=m_cand)
                    return cnt + m_cand.astype(jnp.int32)

                return body

            return one(data_b, one(data_a, zeros)), b0f, cbeff

        cnt, b0, cbef = lax.cond(
            guess_ok, lambda: (cnt, b0, cbef), fallback
        )
        prefix = jnp.full((LANES,), -128, jnp.int32) + b0
        need = need - cbef
        _half_start(g + 1, 0, data_a, sem_a)
        _half_start(g + 1, 1, data_b, sem_b)

        base1 = prefix * 256
        ncand = ((jnp.max(cnt) + 7) // 8) * 8

        def lvl1(i):
            key = cand_v[i]
            valid = i < cnt
            su1 = (key >> 16) - base1
            match = jnp.logical_and(
                valid, jnp.logical_and(su1 >= 0, su1 < 256)
            )
            suc = jnp.clip(su1, 0, 255)
            plsc.addupdate_scatter(hist_v, [suc, lane], ones, mask=match)

        plsc.parallel_loop(0, ncand, unroll=8)(lvl1)
        b1, cbef, _ = _scan_hist(need)
        prefix = base1 + b1
        need = need - cbef
        nrows = ncand

        for shift in (8, 0):
            base = prefix * 256

            def lvlN(i, shift=shift, base=base):
                key = cand_v[i]
                valid = i < cnt
                su = (key >> shift) - base
                match = jnp.logical_and(
                    valid, jnp.logical_and(su >= 0, su < 256)
                )
                suc = jnp.clip(su, 0, 255)
                plsc.addupdate_scatter(hist_v, [suc, lane], ones, mask=match)

            plsc.parallel_loop(0, nrows, unroll=8)(lvlN)
            bN, cbef, _ = _scan_hist(need)
            prefix = base + bN
            need = need - cbef

        tkey = prefix
        rneed = need
        out_off = jnp.full((LANES,), (g % 2) * KTOP, jnp.int32)

        @plsc.parallel_loop(0, nrows, unroll=8, carry=(zeros, zeros))
        def outp(i, st):
            cnt2, eqc = st
            key = cand_v[i]
            valid = i < cnt
            gt = jnp.logical_and(valid, key > tkey)
            eq = jnp.logical_and(valid, key == tkey)
            sel = jnp.logical_or(gt, jnp.logical_and(eq, eqc < rneed))
            val = jnp.where(key < 0, key ^ 0x7FFFFFFF, key)
            plsc.store_scatter(
                outb_v,
                [cnt2 + out_off, lane],
                lax.bitcast_convert_type(val, jnp.float32),
                mask=sel,
            )
            return cnt2 + sel.astype(jnp.int32), eqc + eq.astype(jnp.int32)
        _out_copy(g).start()
        return b0

    lax.fori_loop(
        0, GROUPS_PER_TILE, group_body, jnp.full((LANES,), 192, jnp.int32)
    )

    _out_copy(GROUPS_PER_TILE - 2).wait()
    _out_copy(GROUPS_PER_TILE - 1).wait()

    _half_copy(GROUPS_PER_TILE, 0, data_a, sem_a).wait()
    _half_copy(GROUPS_PER_TILE, 1, data_b, sem_b).wait()


def kernel(x):
    return _kmax_sc(x)
